# baseline (speedup 1.0000x reference)
import functools

import jax
import jax.numpy as jnp
from jax.experimental import pallas as pl
from jax.experimental.pallas import tpu as pltpu


def _fused_kernel(x_ref, w_ref, p_ref, o_ref, y_scr, s_scr,
                  *, H, W, B, Cin, N, eps):
    phase = pl.program_id(1)
    n = pl.program_id(2)
    hw = H * W
    co2 = w_ref.shape[0]

    @pl.when(phase == 0)
    def _conv_phase():
        xb = x_ref[...].reshape(B * Cin, hw).astype(jnp.bfloat16)

        lane = jax.lax.broadcasted_iota(jnp.int32, (1, hw), 1)
        h_idx = lane // W
        w_idx = lane % W

        taps = []
        for dh in (-1, 0, 1):
            for dw in (-1, 0, 1):
                off = dh * W + dw
                shifted = xb if off == 0 else pltpu.roll(xb, (-off) % hw, axis=1)
                if off == 0:
                    taps.append(shifted)
                    continue
                valid = ((h_idx + dh >= 0) & (h_idx + dh < H) &
                         (w_idx + dw >= 0) & (w_idx + dw < W))
                taps.append(jnp.where(valid, shifted, jnp.bfloat16(0)))

        @pl.when(n == 0)
        def _init_stats():
            s_scr[...] = jnp.zeros((co2, 2), jnp.float32)

        ssum = jnp.zeros((co2, 1), jnp.float32)
        ssq = jnp.zeros((co2, 1), jnp.float32)
        for b in range(B):
            x9 = jnp.concatenate([t[b * Cin:(b + 1) * Cin] for t in taps],
                                 axis=0)
            acc = jnp.dot(w_ref[...], x9, preferred_element_type=jnp.float32)
            y_scr[n * B + b] = acc.astype(jnp.bfloat16)
            ssum = ssum + jnp.sum(acc, axis=1, keepdims=True)
            ssq = ssq + jnp.sum(acc * acc, axis=1, keepdims=True)
        s_scr[...] += jnp.concatenate([ssum, ssq], axis=1)

    @pl.when(phase == 1)
    def _bn_phase():
        m = float(N * hw)
        s = s_scr[...]
        mean = s[:, 0:1] * (1.0 / m)
        var = s[:, 1:2] * (1.0 / m) - mean * mean
        inv_std = jax.lax.rsqrt(var + eps)
        scale = p_ref[:, 0:1] * inv_std
        shift = p_ref[:, 1:2] - mean * scale
        alpha = p_ref[:, 2:3]
        for b in range(B):
            y = y_scr[n * B + b].astype(jnp.float32)
            z = y * scale + shift
            o_ref[b] = jnp.where(z >= 0, z, alpha * z)


def kernel(x, w, gamma, beta, alpha, *, eps=1e-5):
    N, Cin, H, W = x.shape
    Cout, Cin_w, KH, KW = w.shape
    assert (KH, KW) == (3, 3) and Cin_w == Cin
    HW = H * W

    B = 4 if N % 4 == 0 else 1
    C_SPLIT = 2 if Cout % 2 == 0 else 1
    Co2 = Cout // C_SPLIT

    x_r = x.reshape(N, Cin, HW)
    w_cat = jnp.transpose(w, (0, 2, 3, 1)).reshape(Cout, 9 * Cin)
    w_cat = w_cat.astype(jnp.bfloat16)
    params = jnp.stack([gamma.astype(jnp.float32), beta.astype(jnp.float32),
                        alpha.astype(jnp.float32)], axis=1)

    out_t = pl.pallas_call(
        functools.partial(_fused_kernel, H=H, W=W, B=B, Cin=Cin, N=N,
                          eps=eps),
        out_shape=jax.ShapeDtypeStruct((N, Cout, HW), jnp.float32),
        grid=(C_SPLIT, 2, N // B),
        in_specs=[
            pl.BlockSpec((B, Cin, HW), lambda c, p, n: (n * (1 - p), 0, 0)),
            pl.BlockSpec((Co2, 9 * Cin), lambda c, p, n: (c, 0)),
            pl.BlockSpec((Co2, 3), lambda c, p, n: (c, 0)),
        ],
        out_specs=pl.BlockSpec((B, Co2, HW), lambda c, p, n: (n * p, c, 0)),
        scratch_shapes=[
            pltpu.VMEM((N, Co2, HW), jnp.bfloat16),
            pltpu.VMEM((Co2, 2), jnp.float32),
        ],
        compiler_params=pltpu.CompilerParams(
            dimension_semantics=("parallel", "arbitrary", "arbitrary")),
    )(x_r, w_cat, params)

    return out_t.reshape(N, Cout, H, W)
